# baseline (speedup 1.0000x reference)
import jax
import jax.numpy as jnp
from jax import lax
from jax.experimental import pallas as pl
from jax.experimental.pallas import tpu as pltpu
from jax.experimental.pallas import tpu_sc as plsc

N = 10000
E = 320000
D = 128
DE = 16

NC = 2
NS = 16
NW = NC * NS
L = 16

CHUNK = 64
QCH = 40
NGROUPS = 4
CHUNKS_PER_TILE = QCH * NGROUPS
EDGES_PER_TILE = CHUNK * CHUNKS_PER_TILE
E_PAD = NW * EDGES_PER_TILE
TOTAL_CHUNKS = NW * CHUNKS_PER_TILE

ROWS_PER_TILE = 632


def _row_slice(sid):
    start = jnp.minimum(sid * ROWS_PER_TILE, N - ROWS_PER_TILE)
    return pl.multiple_of(start, 8)


def _dense_body(x_ref, wt_ref, wb_ref, a2_ref, eat_ref, ae_ref, ab_ref,
                h_ref, sd_ref, base_ref):
    h = jnp.dot(x_ref[...], wt_ref[...], preferred_element_type=jnp.float32)
    h = h + wb_ref[...]
    h_ref[...] = h
    sd_ref[...] = jnp.dot(h, a2_ref[...], preferred_element_type=jnp.float32)
    base_ref[...] = (jnp.dot(ae_ref[...], eat_ref[...],
                             preferred_element_type=jnp.float32)
                     + ab_ref[0, 0])


def _dense(x, wt, wb, a2, eat, ae, ab):
    grid = 10
    nb = N // grid
    eb = E // grid
    return pl.pallas_call(
        _dense_body,
        grid=(grid,),
        in_specs=[
            pl.BlockSpec((nb, D), lambda i: (i, 0)),
            pl.BlockSpec((D, D), lambda i: (0, 0)),
            pl.BlockSpec((1, D), lambda i: (0, 0)),
            pl.BlockSpec((D, 2), lambda i: (0, 0)),
            pl.BlockSpec((DE, eb), lambda i: (0, i)),
            pl.BlockSpec((1, DE), lambda i: (0, 0)),
            pl.BlockSpec((1, 1), lambda i: (0, 0)),
        ],
        out_specs=[
            pl.BlockSpec((nb, D), lambda i: (i, 0)),
            pl.BlockSpec((nb, 2), lambda i: (i, 0)),
            pl.BlockSpec((1, eb), lambda i: (0, i)),
        ],
        out_shape=[
            jax.ShapeDtypeStruct((N, D), jnp.float32),
            jax.ShapeDtypeStruct((N, 2), jnp.float32),
            jax.ShapeDtypeStruct((1, E), jnp.float32),
        ],
    )(x, wt, wb, a2, eat, ae, ab)


def _sc_w_body(sd_hbm, base_hbm, src_hbm, dst_hbm,
               w_hbm, z_hbm,
               sd_v, base_v, src_v, dst_v, w_s, z_v):
    cid = lax.axis_index("c")
    sid = lax.axis_index("s")
    tid = cid * NS + sid

    pltpu.sync_copy(sd_hbm, sd_v)

    def _group(g, zacc):
        gi = tid * NGROUPS + g
        pltpu.sync_copy(base_hbm.at[gi], base_v)
        pltpu.sync_copy(src_hbm.at[gi], src_v)
        pltpu.sync_copy(dst_hbm.at[gi], dst_v)

        def _chunk(c, zacc):
            for j16 in range(CHUNK // L):
                sl = pl.ds(j16 * L, L)
                srci = src_v[c, sl]
                dsti = dst_v[c, sl]
                sv = plsc.load_gather(sd_v, [srci * 2])
                dv = plsc.load_gather(sd_v, [dsti * 2 + 1])
                w16 = jnp.exp(sv + dv + base_v[c, sl])
                w_s[c, sl] = w16
                zacc = zacc + w16
            return zacc
        zacc = lax.fori_loop(0, QCH, _chunk, zacc)
        pltpu.sync_copy(w_s, w_hbm.at[gi])
        return zacc

    zacc = lax.fori_loop(0, NGROUPS, _group, jnp.zeros((L,), jnp.float32))
    z_v[...] = zacc
    pltpu.sync_copy(z_v, z_hbm.at[pl.ds(tid * L, L)])


def _sc_w_call(sd, base3, src3, dst3):
    mesh = plsc.VectorSubcoreMesh(core_axis_name="c", subcore_axis_name="s")
    fn = pl.kernel(
        _sc_w_body,
        out_type=[
            jax.ShapeDtypeStruct((NW * NGROUPS, QCH, CHUNK), jnp.float32),
            jax.ShapeDtypeStruct((NW * L,), jnp.float32),
        ],
        mesh=mesh,
        scratch_types=[
            pltpu.VMEM((2 * N,), jnp.float32),
            pltpu.VMEM((QCH, CHUNK), jnp.float32),
            pltpu.VMEM((QCH, CHUNK), jnp.int32),
            pltpu.VMEM((QCH, CHUNK), jnp.int32),
            pltpu.VMEM((QCH, CHUNK), jnp.float32),
            pltpu.VMEM((L,), jnp.float32),
        ],
        compiler_params=pltpu.CompilerParams(needs_layout_passes=False),
    )
    return fn(sd, base3, src3, dst3)


def _sc_scale_body(src_hbm, w_hbm, h_hbm,
                   rows_hbm,
                   src_v, w_v, g0, g1, s0, s1,
                   h_sh, gsem0, gsem1, ssem0, ssem1):
    cid = lax.axis_index("c")
    sid = lax.axis_index("s")
    tid = cid * NS + sid

    start = _row_slice(sid)
    pltpu.sync_copy(h_hbm.at[pl.ds(start, ROWS_PER_TILE)],
                    h_sh.at[pl.ds(start, ROWS_PER_TILE)])
    plsc.subcore_barrier()

    gbufs = (g0, g1)
    sbufs = (s0, s1)
    gsems = (gsem0, gsem1)
    ssems = (ssem0, ssem1)

    def _issue_gather(c, b):
        pltpu.async_copy(h_sh.at[src_v.at[c]], gbufs[b], gsems[b])

    def _chunk(gi, c, b):
        gbuf, sbuf = gbufs[b], sbufs[b]
        pltpu.make_async_copy(h_sh.at[src_v.at[c]], gbuf, gsems[b]).wait()

        @pl.when(c >= 2)
        def _():
            pltpu.make_async_copy(sbuf, rows_hbm.at[gi * QCH + c - 2],
                                  ssems[b]).wait()

        for j16 in range(CHUNK // L):
            w16 = w_v[c, pl.ds(j16 * L, L)]
            for el in range(L):
                e = j16 * L + el
                wb = jnp.broadcast_to(w16[el], (L,))
                for j in range(D // L):
                    fsl = pl.ds(j * L, L)
                    sbuf[e, fsl] = gbuf[e, fsl] * wb

        pltpu.async_copy(sbuf, rows_hbm.at[gi * QCH + c], ssems[b])

        @pl.when(c + 2 < QCH)
        def _():
            _issue_gather(c + 2, b)

    def _group(g, u):
        gi = tid * NGROUPS + g
        pltpu.sync_copy(src_hbm.at[gi], src_v)
        pltpu.sync_copy(w_hbm.at[gi], w_v)
        _issue_gather(0, 0)
        _issue_gather(1, 1)

        def _pair(p, u):
            _chunk(gi, p * 2, 0)
            _chunk(gi, p * 2 + 1, 1)
            return u
        lax.fori_loop(0, QCH // 2, _pair, 0)

        pltpu.make_async_copy(s0, rows_hbm.at[gi * QCH + QCH - 2],
                              ssem0).wait()
        pltpu.make_async_copy(s1, rows_hbm.at[gi * QCH + QCH - 1],
                              ssem1).wait()
        return u

    lax.fori_loop(0, NGROUPS, _group, 0)


def _sc_scale_call(src3, w3, h):
    mesh = plsc.VectorSubcoreMesh(core_axis_name="c", subcore_axis_name="s")
    fn = pl.kernel(
        _sc_scale_body,
        out_type=jax.ShapeDtypeStruct((TOTAL_CHUNKS, CHUNK, D), jnp.float32),
        mesh=mesh,
        scratch_types=[
            pltpu.VMEM((QCH, CHUNK), jnp.int32),
            pltpu.VMEM((QCH, CHUNK), jnp.float32),
            pltpu.VMEM((CHUNK, D), jnp.float32),
            pltpu.VMEM((CHUNK, D), jnp.float32),
            pltpu.VMEM((CHUNK, D), jnp.float32),
            pltpu.VMEM((CHUNK, D), jnp.float32),
            pltpu.VMEM_SHARED((N, D), jnp.float32),
            pltpu.SemaphoreType.DMA,
            pltpu.SemaphoreType.DMA,
            pltpu.SemaphoreType.DMA,
            pltpu.SemaphoreType.DMA,
        ],
        compiler_params=pltpu.CompilerParams(needs_layout_passes=False),
    )
    return fn(src3, w3, h)


def _sc_scatter_body(dst_hbm, rows_hbm, zrows_hbm,
                     part_hbm,
                     dst_v, r0, r1, r2, r3, acc,
                     rsem0, rsem1, rsem2, rsem3,
                     ssem0, ssem1, ssem2, ssem3):
    cid = lax.axis_index("c")
    sid = lax.axis_index("s")
    tid = cid * NS + sid

    start = _row_slice(sid)
    pltpu.sync_copy(zrows_hbm, acc.at[pl.ds(start, ROWS_PER_TILE)])
    plsc.subcore_barrier()

    rbufs = (r0, r1, r2, r3)
    rsems = (rsem0, rsem1, rsem2, rsem3)
    ssems = (ssem0, ssem1, ssem2, ssem3)

    def _issue_read(gi, c, k):
        pltpu.async_copy(rows_hbm.at[gi * QCH + c], rbufs[k], rsems[k])

    def _chunk(gi, c, k):
        rbuf = rbufs[k]
        k2 = (k + 2) % 4
        pltpu.make_async_copy(rows_hbm.at[gi * QCH + c], rbuf,
                              rsems[k]).wait()
        pltpu.async_copy(rbuf, acc.at[dst_v.at[c]], ssems[k], add=True)

        @pl.when(c >= 2)
        def _():
            pltpu.make_async_copy(rbufs[k2], acc.at[dst_v.at[c - 2]],
                                  ssems[k2]).wait()

        @pl.when(c + 2 < QCH)
        def _():
            _issue_read(gi, c + 2, k2)

    def _group(g, u):
        gi = tid * NGROUPS + g
        pltpu.sync_copy(dst_hbm.at[gi], dst_v)
        _issue_read(gi, 0, 0)
        _issue_read(gi, 1, 1)

        def _quad(p, u):
            for k in range(4):
                _chunk(gi, p * 4 + k, k)
            return u
        lax.fori_loop(0, QCH // 4, _quad, 0)

        k_a = (QCH - 2) % 4
        k_b = (QCH - 1) % 4
        pltpu.make_async_copy(rbufs[k_a], acc.at[dst_v.at[QCH - 2]],
                              ssems[k_a]).wait()
        pltpu.make_async_copy(rbufs[k_b], acc.at[dst_v.at[QCH - 1]],
                              ssems[k_b]).wait()
        return u

    lax.fori_loop(0, NGROUPS, _group, 0)

    plsc.subcore_barrier()
    pltpu.sync_copy(acc.at[pl.ds(start, ROWS_PER_TILE)],
                    part_hbm.at[cid, pl.ds(start, ROWS_PER_TILE)])


def _sc_scatter_call(dst3, rows, zrows):
    mesh = plsc.VectorSubcoreMesh(core_axis_name="c", subcore_axis_name="s")
    fn = pl.kernel(
        _sc_scatter_body,
        out_type=jax.ShapeDtypeStruct((NC, N, D), jnp.float32),
        mesh=mesh,
        scratch_types=[
            pltpu.VMEM((QCH, CHUNK), jnp.int32),
            pltpu.VMEM((CHUNK, D), jnp.float32),
            pltpu.VMEM((CHUNK, D), jnp.float32),
            pltpu.VMEM((CHUNK, D), jnp.float32),
            pltpu.VMEM((CHUNK, D), jnp.float32),
            pltpu.VMEM_SHARED((N, D), jnp.float32),
            pltpu.SemaphoreType.DMA,
            pltpu.SemaphoreType.DMA,
            pltpu.SemaphoreType.DMA,
            pltpu.SemaphoreType.DMA,
            pltpu.SemaphoreType.DMA,
            pltpu.SemaphoreType.DMA,
            pltpu.SemaphoreType.DMA,
            pltpu.SemaphoreType.DMA,
        ],
        compiler_params=pltpu.CompilerParams(needs_layout_passes=False),
    )
    return fn(dst3, rows, zrows)


def _combine_body(part_ref, z_ref, out_ref):
    z = jnp.sum(z_ref[...])
    p = part_ref[...]
    out_ref[...] = (p[0] + p[1]) * (1.0 / z)


def _combine(part, zs):
    grid = 10
    nb = N // grid
    return pl.pallas_call(
        _combine_body,
        grid=(grid,),
        in_specs=[
            pl.BlockSpec((NC, nb, D), lambda i: (0, i, 0)),
            pl.BlockSpec((NW * L,), lambda i: (0,)),
        ],
        out_specs=pl.BlockSpec((nb, D), lambda i: (i, 0)),
        out_shape=jax.ShapeDtypeStruct((N, D), jnp.float32),
    )(part, zs)


def kernel(x, edge_index, edge_attr, W_w, W_b, A_w, A_b):
    wt = W_w.T
    wb = W_b.reshape(1, D)
    a_src = A_w[0, :D]
    a_dst = A_w[0, D:2 * D]
    a2 = jnp.stack([a_src, a_dst], axis=1)
    eat = edge_attr.T
    ae = A_w[0, 2 * D:].reshape(1, DE)
    ab = A_b.reshape(1, 1)

    h, sd, base2 = _dense(x, wt, wb, a2, eat, ae, ab)

    base = base2.reshape(E)
    pad = E_PAD - E
    neg = jnp.full((pad,), -1e30, jnp.float32)
    base3 = jnp.concatenate([base, neg]).reshape(NW * NGROUPS, QCH, CHUNK)
    zpad = jnp.zeros((pad,), jnp.int32)
    src3 = jnp.concatenate([edge_index[0], zpad]).reshape(
        NW * NGROUPS, QCH, CHUNK)
    dst3 = jnp.concatenate([edge_index[1], zpad]).reshape(
        NW * NGROUPS, QCH, CHUNK)
    sdflat = sd.reshape(2 * N)
    zrows = jnp.zeros((ROWS_PER_TILE, D), jnp.float32)

    w3, zs = _sc_w_call(sdflat, base3, src3, dst3)
    rows = _sc_scale_call(src3, w3, h)
    part = _sc_scatter_call(dst3, rows, zrows)
    return _combine(part, zs)
